# Initial kernel scaffold; baseline (speedup 1.0000x reference)
import jax
import jax.numpy as jnp
from jax.experimental import pallas as pl


def kernel(x, edge_index, batch, W1, b1, W2, b2, Wf, bf):
    raise NotImplementedError("write your pallas kernel here")


# speedup vs baseline: 13.2193x; 13.2193x over previous
import functools

import jax
import jax.numpy as jnp
from jax import lax
from jax.experimental import pallas as pl
from jax.experimental.pallas import tpu as pltpu
from jax.experimental.pallas import tpu_sc as plsc

_N = 10000
_D = 128
_E = 320000
_G = 64

_NC = 2
_NS = 16
_EP = _E // (_NC * _NS)
_K = 80
_NCH = _EP // _K
_RPT = 624
_ZCH = _RPT // 16

_BLK = 1000
_NBLK = _N // _BLK


def _sc_mesh():
    return plsc.VectorSubcoreMesh(core_axis_name="c", subcore_axis_name="s",
                                  num_cores=_NC, num_subcores=_NS)


def _sc_degree_body(dst_hbm, out_hbm, didx, ones, zbuf, fbuf, dacc):
    c = lax.axis_index("c")
    s = lax.axis_index("s")
    row0 = s * _RPT
    for q in range(16):
        zbuf[pl.ds(q * 16, 16)] = jnp.zeros((16,), jnp.float32)
    for q in range(_K // 16):
        ones[pl.ds(q * 16, 16)] = jnp.ones((16,), jnp.float32)

    def _zero(i, carry):
        pltpu.sync_copy(zbuf, dacc.at[pl.ds(row0 + i * 256, 256)])
        return carry

    lax.fori_loop(0, _RPT // 256, _zero, 0)
    pltpu.sync_copy(zbuf.at[pl.ds(0, _RPT % 256)],
                    dacc.at[pl.ds(row0 + _RPT - _RPT % 256, _RPT % 256)])

    @pl.when(s == _NS - 1)
    def _zero_tail():
        pltpu.sync_copy(zbuf.at[pl.ds(0, 16)], dacc.at[pl.ds(_N - 16, 16)])

    plsc.subcore_barrier()

    base = (c * _NS + s) * _EP

    def _body(i, carry):
        pltpu.sync_copy(dst_hbm.at[pl.ds(base + i * _K, _K)], didx)
        pltpu.sync_copy(ones, dacc.at[didx], add=True)
        return carry

    lax.fori_loop(0, _NCH, _body, 0)
    plsc.subcore_barrier()
    pltpu.sync_copy(dacc.at[pl.ds(row0, _RPT)], fbuf.at[pl.ds(0, _RPT)])
    pltpu.sync_copy(fbuf.at[pl.ds(0, _RPT)],
                    out_hbm.at[pl.ds(c * _N + row0, _RPT)])

    @pl.when(s == _NS - 1)
    def _flush_tail():
        pltpu.sync_copy(dacc.at[pl.ds(_NS * _RPT, _N - _NS * _RPT)],
                        fbuf.at[pl.ds(0, _N - _NS * _RPT)])
        pltpu.sync_copy(fbuf.at[pl.ds(0, _N - _NS * _RPT)],
                        out_hbm.at[pl.ds(c * _N + _NS * _RPT,
                                         _N - _NS * _RPT)])


def _sc_edge_scatter_body(y_hbm, src_hbm, dst_hbm, out_hbm,
                          sidx, didx, rows, zbuf, zacc, sem):
    c = lax.axis_index("c")
    s = lax.axis_index("s")
    row0 = s * _RPT
    for r in range(16):
        for q in range(_D // 16):
            zbuf[r, pl.ds(q * 16, 16)] = jnp.zeros((16,), jnp.float32)

    def _zero(i, carry):
        pltpu.sync_copy(zbuf, zacc.at[pl.ds(row0 + i * 16, 16)])
        return carry

    lax.fori_loop(0, _ZCH, _zero, 0)

    @pl.when(s == _NS - 1)
    def _zero_tail():
        pltpu.sync_copy(zbuf, zacc.at[pl.ds(_N - 16, 16)])

    plsc.subcore_barrier()

    base = (c * _NS + s) * _EP

    def _body(i, carry):
        off = base + i * _K
        pltpu.sync_copy(src_hbm.at[pl.ds(off, _K)], sidx)
        pltpu.sync_copy(dst_hbm.at[pl.ds(off, _K)], didx)
        pltpu.async_copy(y_hbm.at[sidx], rows, sem).wait()
        pltpu.sync_copy(rows, zacc.at[didx], add=True)
        return carry

    lax.fori_loop(0, _NCH, _body, 0)
    plsc.subcore_barrier()
    pltpu.sync_copy(zacc.at[pl.ds(row0, _RPT)],
                    out_hbm.at[pl.ds(c * _N + row0, _RPT)])

    @pl.when(s == _NS - 1)
    def _flush_tail():
        pltpu.sync_copy(zacc.at[pl.ds(_NS * _RPT, _N - _NS * _RPT)],
                        out_hbm.at[pl.ds(c * _N + _NS * _RPT,
                                         _N - _NS * _RPT)])


@functools.cache
def _sc_kernels():
    sc_degree = pl.kernel(
        _sc_degree_body,
        out_type=jax.ShapeDtypeStruct((_NC * _N,), jnp.float32),
        mesh=_sc_mesh(),
        scratch_types=[
            pltpu.VMEM((_K,), jnp.int32),
            pltpu.VMEM((_K,), jnp.float32),
            pltpu.VMEM((256,), jnp.float32),
            pltpu.VMEM((_RPT,), jnp.float32),
            pltpu.VMEM_SHARED((_N,), jnp.float32),
        ],
    )
    sc_edge_scatter = pl.kernel(
        _sc_edge_scatter_body,
        out_type=jax.ShapeDtypeStruct((_NC * _N, _D), jnp.float32),
        mesh=_sc_mesh(),
        scratch_types=[
            pltpu.VMEM((_K,), jnp.int32),
            pltpu.VMEM((_K,), jnp.int32),
            pltpu.VMEM((_K, _D), jnp.float32),
            pltpu.VMEM((16, _D), jnp.float32),
            pltpu.VMEM_SHARED((_N, _D), jnp.float32),
            pltpu.SemaphoreType.DMA,
        ],
    )
    return sc_degree, sc_edge_scatter


def _dinv(dA_ref, dB_ref):
    deg = dA_ref[...] + dB_ref[...] + 1.0
    return lax.rsqrt(jnp.maximum(deg, 1.0))


def _mm_scale_body(x_ref, dA_ref, dB_ref, w_ref, y_ref):
    y_ref[...] = _dinv(dA_ref, dB_ref) * jnp.dot(
        x_ref[...], w_ref[...], preferred_element_type=jnp.float32)


def _mm_scale(x, degp, w):
    return pl.pallas_call(
        _mm_scale_body,
        grid=(_NBLK,),
        in_specs=[
            pl.BlockSpec((_BLK, _D), lambda i: (i, 0)),
            pl.BlockSpec((_BLK, 1), lambda i: (i, 0)),
            pl.BlockSpec((_BLK, 1), lambda i: (i + _NBLK, 0)),
            pl.BlockSpec((_D, _D), lambda i: (0, 0)),
        ],
        out_specs=pl.BlockSpec((_BLK, _D), lambda i: (i, 0)),
        out_shape=jax.ShapeDtypeStruct((_N, _D), jnp.float32),
    )(x, degp, degp, w)


def _fuse_body(zA_ref, zB_ref, y_ref, dA_ref, dB_ref, b_ref, w_ref, o_ref):
    dinv = _dinv(dA_ref, dB_ref)
    sconv = zA_ref[...] + zB_ref[...] + y_ref[...]
    h = jnp.maximum(dinv * sconv + b_ref[...], 0.0)
    o_ref[...] = dinv * jnp.dot(h, w_ref[...],
                                preferred_element_type=jnp.float32)


def _fuse(z, y, degp, b, w):
    return pl.pallas_call(
        _fuse_body,
        grid=(_NBLK,),
        in_specs=[
            pl.BlockSpec((_BLK, _D), lambda i: (i, 0)),
            pl.BlockSpec((_BLK, _D), lambda i: (i + _NBLK, 0)),
            pl.BlockSpec((_BLK, _D), lambda i: (i, 0)),
            pl.BlockSpec((_BLK, 1), lambda i: (i, 0)),
            pl.BlockSpec((_BLK, 1), lambda i: (i + _NBLK, 0)),
            pl.BlockSpec((1, _D), lambda i: (0, 0)),
            pl.BlockSpec((_D, _D), lambda i: (0, 0)),
        ],
        out_specs=pl.BlockSpec((_BLK, _D), lambda i: (i, 0)),
        out_shape=jax.ShapeDtypeStruct((_N, _D), jnp.float32),
    )(z, z, y, degp, degp, b, w)


def _final_body(zA_ref, zB_ref, y_ref, dA_ref, dB_ref, b_ref, bt_ref,
                wf_ref, bf_ref, o_ref, acc, cnt):
    i = pl.program_id(0)

    @pl.when(i == 0)
    def _():
        acc[...] = jnp.zeros_like(acc)
        cnt[...] = jnp.zeros_like(cnt)

    dinv = _dinv(dA_ref, dB_ref)
    sconv = zA_ref[...] + zB_ref[...] + y_ref[...]
    h = jnp.maximum(dinv * sconv + b_ref[...], 0.0)
    pt = (bt_ref[...] == lax.broadcasted_iota(
        jnp.int32, (_BLK, _G), 1)).astype(jnp.float32)
    dn = (((0,), (0,)), ((), ()))
    acc[...] += lax.dot_general(pt, h, dn,
                                preferred_element_type=jnp.float32)
    cnt[...] += lax.dot_general(pt, jnp.ones_like(h), dn,
                                preferred_element_type=jnp.float32)

    @pl.when(i == _NBLK - 1)
    def _():
        pooled = acc[...] / jnp.maximum(cnt[...], 1.0)
        o_ref[...] = jnp.maximum(
            jnp.dot(pooled, wf_ref[...],
                    preferred_element_type=jnp.float32) + bf_ref[...], 0.0)


def _final(z, y, degp, b, bt, wf, bf):
    return pl.pallas_call(
        _final_body,
        grid=(_NBLK,),
        in_specs=[
            pl.BlockSpec((_BLK, _D), lambda i: (i, 0)),
            pl.BlockSpec((_BLK, _D), lambda i: (i + _NBLK, 0)),
            pl.BlockSpec((_BLK, _D), lambda i: (i, 0)),
            pl.BlockSpec((_BLK, 1), lambda i: (i, 0)),
            pl.BlockSpec((_BLK, 1), lambda i: (i + _NBLK, 0)),
            pl.BlockSpec((1, _D), lambda i: (0, 0)),
            pl.BlockSpec((_BLK, 1), lambda i: (i, 0)),
            pl.BlockSpec((_D, _D), lambda i: (0, 0)),
            pl.BlockSpec((1, _D), lambda i: (0, 0)),
        ],
        out_specs=pl.BlockSpec((_G, _D), lambda i: (0, 0)),
        out_shape=jax.ShapeDtypeStruct((_G, _D), jnp.float32),
        scratch_shapes=[
            pltpu.VMEM((_G, _D), jnp.float32),
            pltpu.VMEM((_G, _D), jnp.float32),
        ],
    )(z, z, y, degp, degp, b, bt, wf, bf)


def kernel(x, edge_index, batch, W1, b1, W2, b2, Wf, bf):
    sc_degree, sc_edge_scatter = _sc_kernels()
    src = edge_index[0].astype(jnp.int32)
    dst = edge_index[1].astype(jnp.int32)
    degp = sc_degree(dst).reshape(_NC * _N, 1)
    y1 = _mm_scale(x, degp, W1)
    z1 = sc_edge_scatter(y1, src, dst)
    y2 = _fuse(z1, y1, degp, b1.reshape(1, _D), W2)
    z2 = sc_edge_scatter(y2, src, dst)
    return _final(z2, y2, degp, b2.reshape(1, _D),
                  batch.reshape(_N, 1).astype(jnp.int32),
                  Wf, bf.reshape(1, _D))


# speedup vs baseline: 26.2340x; 1.9845x over previous
import functools

import jax
import jax.numpy as jnp
from jax import lax
from jax.experimental import pallas as pl
from jax.experimental.pallas import tpu as pltpu
from jax.experimental.pallas import tpu_sc as plsc

_N = 10000
_D = 128
_E = 320000
_G = 64

_NC = 2
_NS = 16
_EP = _E // (_NC * _NS)
_K = 80
_NCH = _EP // _K
_KE = 80
_NCHE = _EP // _KE
_RPT = 624
_ZCH = _RPT // 16

_BLK = 1000
_NBLK = _N // _BLK


def _sc_mesh():
    return plsc.VectorSubcoreMesh(core_axis_name="c", subcore_axis_name="s",
                                  num_cores=_NC, num_subcores=_NS)


def _sc_degree_body(dst_hbm, out_hbm, didx, ones, zbuf, fbuf, dacc):
    c = lax.axis_index("c")
    s = lax.axis_index("s")
    row0 = s * _RPT
    for q in range(16):
        zbuf[pl.ds(q * 16, 16)] = jnp.zeros((16,), jnp.float32)
    for q in range(_K // 16):
        ones[pl.ds(q * 16, 16)] = jnp.ones((16,), jnp.float32)

    def _zero(i, carry):
        pltpu.sync_copy(zbuf, dacc.at[pl.ds(row0 + i * 256, 256)])
        return carry

    lax.fori_loop(0, _RPT // 256, _zero, 0)
    pltpu.sync_copy(zbuf.at[pl.ds(0, _RPT % 256)],
                    dacc.at[pl.ds(row0 + _RPT - _RPT % 256, _RPT % 256)])

    @pl.when(s == _NS - 1)
    def _zero_tail():
        pltpu.sync_copy(zbuf.at[pl.ds(0, 16)], dacc.at[pl.ds(_N - 16, 16)])

    plsc.subcore_barrier()

    base = (c * _NS + s) * _EP

    def _body(i, carry):
        pltpu.sync_copy(dst_hbm.at[pl.ds(base + i * _K, _K)], didx)
        pltpu.sync_copy(ones, dacc.at[didx], add=True)
        return carry

    lax.fori_loop(0, _NCH, _body, 0)
    plsc.subcore_barrier()
    pltpu.sync_copy(dacc.at[pl.ds(row0, _RPT)], fbuf.at[pl.ds(0, _RPT)])
    pltpu.sync_copy(fbuf.at[pl.ds(0, _RPT)],
                    out_hbm.at[pl.ds(c * _N + row0, _RPT)])

    @pl.when(s == _NS - 1)
    def _flush_tail():
        pltpu.sync_copy(dacc.at[pl.ds(_NS * _RPT, _N - _NS * _RPT)],
                        fbuf.at[pl.ds(0, _N - _NS * _RPT)])
        pltpu.sync_copy(fbuf.at[pl.ds(0, _N - _NS * _RPT)],
                        out_hbm.at[pl.ds(c * _N + _NS * _RPT,
                                         _N - _NS * _RPT)])


def _sc_edge_scatter_body(y_hbm, src_hbm, dst_hbm, out_hbm,
                          sidx, didx, rows0, rows1, zbuf, zacc,
                          sem0, sem1):
    c = lax.axis_index("c")
    s = lax.axis_index("s")
    row0 = s * _RPT
    for r in range(16):
        for q in range(_D // 16):
            zbuf[r, pl.ds(q * 16, 16)] = jnp.zeros((16,), jnp.float32)

    def _zero(i, carry):
        pltpu.sync_copy(zbuf, zacc.at[pl.ds(row0 + i * 16, 16)])
        return carry

    lax.fori_loop(0, _ZCH, _zero, 0)

    @pl.when(s == _NS - 1)
    def _zero_tail():
        pltpu.sync_copy(zbuf, zacc.at[pl.ds(_N - 16, 16)])

    plsc.subcore_barrier()

    wid = c * _NS + s
    pltpu.sync_copy(src_hbm.at[pl.ds(wid * _EP, _EP)], sidx)
    pltpu.sync_copy(dst_hbm.at[wid], didx)

    def _sview(j):
        return sidx.at[pl.ds(j * _KE, _KE)]

    pltpu.async_copy(y_hbm.at[_sview(0)], rows0, sem0)

    def _body(t, carry):
        j = 2 * t
        pltpu.async_copy(y_hbm.at[_sview(j + 1)], rows1, sem1)
        pltpu.make_async_copy(y_hbm.at[_sview(j)], rows0, sem0).wait()
        pltpu.sync_copy(rows0, zacc.at[didx.at[j]], add=True)
        pltpu.async_copy(y_hbm.at[_sview(j + 2)], rows0, sem0)
        pltpu.make_async_copy(y_hbm.at[_sview(j + 1)], rows1, sem1).wait()
        pltpu.sync_copy(rows1, zacc.at[didx.at[j + 1]], add=True)
        return carry

    lax.fori_loop(0, (_NCHE - 1) // 2, _body, 0)
    pltpu.make_async_copy(y_hbm.at[_sview(_NCHE - 1)], rows0, sem0).wait()
    pltpu.sync_copy(rows0, zacc.at[didx.at[_NCHE - 1]], add=True)
    plsc.subcore_barrier()
    pltpu.sync_copy(zacc.at[pl.ds(row0, _RPT)],
                    out_hbm.at[pl.ds(c * _N + row0, _RPT)])

    @pl.when(s == _NS - 1)
    def _flush_tail():
        pltpu.sync_copy(zacc.at[pl.ds(_NS * _RPT, _N - _NS * _RPT)],
                        out_hbm.at[pl.ds(c * _N + _NS * _RPT,
                                         _N - _NS * _RPT)])


@functools.cache
def _sc_kernels():
    sc_degree = pl.kernel(
        _sc_degree_body,
        out_type=jax.ShapeDtypeStruct((_NC * _N,), jnp.float32),
        mesh=_sc_mesh(),
        scratch_types=[
            pltpu.VMEM((_K,), jnp.int32),
            pltpu.VMEM((_K,), jnp.float32),
            pltpu.VMEM((256,), jnp.float32),
            pltpu.VMEM((_RPT,), jnp.float32),
            pltpu.VMEM_SHARED((_N,), jnp.float32),
        ],
    )
    sc_edge_scatter = pl.kernel(
        _sc_edge_scatter_body,
        out_type=jax.ShapeDtypeStruct((_NC * _N, _D), jnp.float32),
        mesh=_sc_mesh(),
        scratch_types=[
            pltpu.VMEM((_EP,), jnp.int32),
            pltpu.VMEM((_NCHE, _KE), jnp.int32),
            pltpu.VMEM((_KE, _D), jnp.float32),
            pltpu.VMEM((_KE, _D), jnp.float32),
            pltpu.VMEM((16, _D), jnp.float32),
            pltpu.VMEM_SHARED((_N, _D), jnp.float32),
            pltpu.SemaphoreType.DMA,
            pltpu.SemaphoreType.DMA,
        ],
    )
    return sc_degree, sc_edge_scatter


def _dinv(dA_ref, dB_ref):
    deg = dA_ref[...] + dB_ref[...] + 1.0
    return lax.rsqrt(jnp.maximum(deg, 1.0))


def _mm_scale_body(x_ref, dA_ref, dB_ref, w_ref, y_ref):
    y_ref[...] = _dinv(dA_ref, dB_ref) * jnp.dot(
        x_ref[...], w_ref[...], preferred_element_type=jnp.float32)


def _mm_scale(x, degp, w):
    return pl.pallas_call(
        _mm_scale_body,
        grid=(_NBLK,),
        in_specs=[
            pl.BlockSpec((_BLK, _D), lambda i: (i, 0)),
            pl.BlockSpec((_BLK, 1), lambda i: (i, 0)),
            pl.BlockSpec((_BLK, 1), lambda i: (i + _NBLK, 0)),
            pl.BlockSpec((_D, _D), lambda i: (0, 0)),
        ],
        out_specs=pl.BlockSpec((_BLK, _D), lambda i: (i, 0)),
        out_shape=jax.ShapeDtypeStruct((_N, _D), jnp.float32),
    )(x, degp, degp, w)


def _fuse_body(zA_ref, zB_ref, y_ref, dA_ref, dB_ref, b_ref, w_ref, o_ref):
    dinv = _dinv(dA_ref, dB_ref)
    sconv = zA_ref[...] + zB_ref[...] + y_ref[...]
    h = jnp.maximum(dinv * sconv + b_ref[...], 0.0)
    o_ref[...] = dinv * jnp.dot(h, w_ref[...],
                                preferred_element_type=jnp.float32)


def _fuse(z, y, degp, b, w):
    return pl.pallas_call(
        _fuse_body,
        grid=(_NBLK,),
        in_specs=[
            pl.BlockSpec((_BLK, _D), lambda i: (i, 0)),
            pl.BlockSpec((_BLK, _D), lambda i: (i + _NBLK, 0)),
            pl.BlockSpec((_BLK, _D), lambda i: (i, 0)),
            pl.BlockSpec((_BLK, 1), lambda i: (i, 0)),
            pl.BlockSpec((_BLK, 1), lambda i: (i + _NBLK, 0)),
            pl.BlockSpec((1, _D), lambda i: (0, 0)),
            pl.BlockSpec((_D, _D), lambda i: (0, 0)),
        ],
        out_specs=pl.BlockSpec((_BLK, _D), lambda i: (i, 0)),
        out_shape=jax.ShapeDtypeStruct((_N, _D), jnp.float32),
    )(z, z, y, degp, degp, b, w)


def _final_body(zA_ref, zB_ref, y_ref, dA_ref, dB_ref, b_ref, bt_ref,
                wf_ref, bf_ref, o_ref, acc, cnt):
    i = pl.program_id(0)

    @pl.when(i == 0)
    def _():
        acc[...] = jnp.zeros_like(acc)
        cnt[...] = jnp.zeros_like(cnt)

    dinv = _dinv(dA_ref, dB_ref)
    sconv = zA_ref[...] + zB_ref[...] + y_ref[...]
    h = jnp.maximum(dinv * sconv + b_ref[...], 0.0)
    pt = (bt_ref[...] == lax.broadcasted_iota(
        jnp.int32, (_BLK, _G), 1)).astype(jnp.float32)
    dn = (((0,), (0,)), ((), ()))
    acc[...] += lax.dot_general(pt, h, dn,
                                preferred_element_type=jnp.float32)
    cnt[...] += lax.dot_general(pt, jnp.ones_like(h), dn,
                                preferred_element_type=jnp.float32)

    @pl.when(i == _NBLK - 1)
    def _():
        pooled = acc[...] / jnp.maximum(cnt[...], 1.0)
        o_ref[...] = jnp.maximum(
            jnp.dot(pooled, wf_ref[...],
                    preferred_element_type=jnp.float32) + bf_ref[...], 0.0)


def _final(z, y, degp, b, bt, wf, bf):
    return pl.pallas_call(
        _final_body,
        grid=(_NBLK,),
        in_specs=[
            pl.BlockSpec((_BLK, _D), lambda i: (i, 0)),
            pl.BlockSpec((_BLK, _D), lambda i: (i + _NBLK, 0)),
            pl.BlockSpec((_BLK, _D), lambda i: (i, 0)),
            pl.BlockSpec((_BLK, 1), lambda i: (i, 0)),
            pl.BlockSpec((_BLK, 1), lambda i: (i + _NBLK, 0)),
            pl.BlockSpec((1, _D), lambda i: (0, 0)),
            pl.BlockSpec((_BLK, 1), lambda i: (i, 0)),
            pl.BlockSpec((_D, _D), lambda i: (0, 0)),
            pl.BlockSpec((1, _D), lambda i: (0, 0)),
        ],
        out_specs=pl.BlockSpec((_G, _D), lambda i: (0, 0)),
        out_shape=jax.ShapeDtypeStruct((_G, _D), jnp.float32),
        scratch_shapes=[
            pltpu.VMEM((_G, _D), jnp.float32),
            pltpu.VMEM((_G, _D), jnp.float32),
        ],
    )(z, z, y, degp, degp, b, bt, wf, bf)


def kernel(x, edge_index, batch, W1, b1, W2, b2, Wf, bf):
    sc_degree, sc_edge_scatter = _sc_kernels()
    src = edge_index[0].astype(jnp.int32)
    dst = edge_index[1].astype(jnp.int32)
    dst2 = dst.reshape(_NC * _NS, _NCHE, _KE)
    degp = sc_degree(dst).reshape(_NC * _N, 1)
    y1 = _mm_scale(x, degp, W1)
    z1 = sc_edge_scatter(y1, src, dst2)
    y2 = _fuse(z1, y1, degp, b1.reshape(1, _D), W2)
    z2 = sc_edge_scatter(y2, src, dst2)
    return _final(z2, y2, degp, b2.reshape(1, _D),
                  batch.reshape(_N, 1).astype(jnp.int32),
                  Wf, bf.reshape(1, _D))


# speedup vs baseline: 30.0427x; 1.1452x over previous
import functools

import jax
import jax.numpy as jnp
from jax import lax
from jax.experimental import pallas as pl
from jax.experimental.pallas import tpu as pltpu
from jax.experimental.pallas import tpu_sc as plsc

_N = 10000
_D = 128
_E = 320000
_G = 64

_NC = 2
_NS = 16
_EP = _E // (_NC * _NS)
_K = 80
_NCH = _EP // _K
_KE = 80
_NCHE = _EP // _KE
_PH0 = 64
_RPT = 624
_ZCH = _RPT // 16

_BLK = 1000
_NBLK = _N // _BLK


def _sc_mesh():
    return plsc.VectorSubcoreMesh(core_axis_name="c", subcore_axis_name="s",
                                  num_cores=_NC, num_subcores=_NS)


def _sc_degree_body(dst_hbm, out_hbm, didx, ones, zbuf, fbuf, dacc, sem):
    c = lax.axis_index("c")
    s = lax.axis_index("s")
    row0 = s * _RPT
    for q in range(16):
        zbuf[pl.ds(q * 16, 16)] = jnp.zeros((16,), jnp.float32)
    for q in range(_K // 16):
        ones[pl.ds(q * 16, 16)] = jnp.ones((16,), jnp.float32)

    def _zero(i, carry):
        pltpu.sync_copy(zbuf, dacc.at[pl.ds(row0 + i * 256, 256)])
        return carry

    lax.fori_loop(0, _RPT // 256, _zero, 0)
    pltpu.sync_copy(zbuf.at[pl.ds(0, _RPT % 256)],
                    dacc.at[pl.ds(row0 + _RPT - _RPT % 256, _RPT % 256)])

    @pl.when(s == _NS - 1)
    def _zero_tail():
        pltpu.sync_copy(zbuf.at[pl.ds(0, 16)], dacc.at[pl.ds(_N - 16, 16)])

    plsc.subcore_barrier()

    wid = c * _NS + s
    pltpu.sync_copy(dst_hbm.at[wid], didx)

    def _fire(i, carry):
        pltpu.async_copy(ones, dacc.at[didx.at[i]], sem, add=True)
        return carry

    lax.fori_loop(0, _NCH, _fire, 0)

    def _drain(i, carry):
        pltpu.make_async_copy(ones, dacc.at[didx.at[0]], sem).wait()
        return carry

    lax.fori_loop(0, _NCH, _drain, 0)
    plsc.subcore_barrier()
    pltpu.sync_copy(dacc.at[pl.ds(row0, _RPT)], fbuf.at[pl.ds(0, _RPT)])
    pltpu.sync_copy(fbuf.at[pl.ds(0, _RPT)],
                    out_hbm.at[pl.ds(c * _N + row0, _RPT)])

    @pl.when(s == _NS - 1)
    def _flush_tail():
        pltpu.sync_copy(dacc.at[pl.ds(_NS * _RPT, _N - _NS * _RPT)],
                        fbuf.at[pl.ds(0, _N - _NS * _RPT)])
        pltpu.sync_copy(fbuf.at[pl.ds(0, _N - _NS * _RPT)],
                        out_hbm.at[pl.ds(c * _N + _NS * _RPT,
                                         _N - _NS * _RPT)])


def _sc_edge_scatter_body(y_hbm, src_hbm, dst_hbm, out_hbm,
                          sidx, didx, rows0, rows1, rows2, zbuf, zacc,
                          sg0, sg1, sg2, ss0, ss1, ss2):
    c = lax.axis_index("c")
    s = lax.axis_index("s")
    row0 = s * _RPT
    for r in range(16):
        for q in range(_D // 16):
            zbuf[r, pl.ds(q * 16, 16)] = jnp.zeros((16,), jnp.float32)

    def _zero(i, carry):
        pltpu.sync_copy(zbuf, zacc.at[pl.ds(row0 + i * 16, 16)])
        return carry

    lax.fori_loop(0, _ZCH, _zero, 0)

    @pl.when(s == _NS - 1)
    def _zero_tail():
        pltpu.sync_copy(zbuf, zacc.at[pl.ds(_N - 16, 16)])

    plsc.subcore_barrier()

    wid = c * _NS + s
    rows_ = (rows0, rows1, rows2)
    sg = (sg0, sg1, sg2)
    ss = (ss0, ss1, ss2)

    def _gather(j, b):
        pltpu.async_copy(y_hbm.at[sidx.at[pl.ds(j * _KE, _KE)]],
                         rows_[b], sg[b])

    def _wait_g(b):
        pltpu.make_async_copy(y_hbm.at[sidx.at[pl.ds(0, _KE)]],
                              rows_[b], sg[b]).wait()

    def _scatter(j, b):
        pltpu.async_copy(rows_[b], zacc.at[didx.at[j]], ss[b], add=True)

    def _wait_s(b):
        pltpu.make_async_copy(rows_[b], zacc.at[didx.at[0]], ss[b]).wait()

    for ph, nch in ((0, _PH0), (1, _NCHE - _PH0)):
        g0 = ph * _PH0
        pltpu.sync_copy(src_hbm.at[pl.ds(wid * _EP + g0 * _KE, nch * _KE)],
                        sidx.at[pl.ds(0, nch * _KE)])
        pltpu.sync_copy(dst_hbm.at[wid, pl.ds(g0, nch)],
                        didx.at[pl.ds(0, nch)])
        nt = nch // 3
        for b in range(3):
            _gather(b, b)
        for b in range(3):
            _wait_g(b)
            _scatter(b, b)

        def _body(t, carry):
            j = 3 * t
            for b in range(3):
                _wait_s(b)
                _gather(j + b, b)
            for b in range(3):
                _wait_g(b)
                _scatter(j + b, b)
            return carry

        lax.fori_loop(1, nt, _body, 0)
        for j in range(3 * nt, nch):
            b = j - 3 * nt
            _wait_s(b)
            _gather(j, b)
            _wait_g(b)
            _scatter(j, b)
        for b in range(3):
            _wait_s(b)

    plsc.subcore_barrier()
    pltpu.sync_copy(zacc.at[pl.ds(row0, _RPT)],
                    out_hbm.at[pl.ds(c * _N + row0, _RPT)])

    @pl.when(s == _NS - 1)
    def _flush_tail():
        pltpu.sync_copy(zacc.at[pl.ds(_NS * _RPT, _N - _NS * _RPT)],
                        out_hbm.at[pl.ds(c * _N + _NS * _RPT,
                                         _N - _NS * _RPT)])


@functools.cache
def _sc_kernels():
    sc_degree = pl.kernel(
        _sc_degree_body,
        out_type=jax.ShapeDtypeStruct((_NC * _N,), jnp.float32),
        mesh=_sc_mesh(),
        scratch_types=[
            pltpu.VMEM((_NCH, _K), jnp.int32),
            pltpu.VMEM((_K,), jnp.float32),
            pltpu.VMEM((256,), jnp.float32),
            pltpu.VMEM((_RPT,), jnp.float32),
            pltpu.VMEM_SHARED((_N,), jnp.float32),
            pltpu.SemaphoreType.DMA,
        ],
    )
    sc_edge_scatter = pl.kernel(
        _sc_edge_scatter_body,
        out_type=jax.ShapeDtypeStruct((_NC * _N, _D), jnp.float32),
        mesh=_sc_mesh(),
        scratch_types=[
            pltpu.VMEM((_PH0 * _KE,), jnp.int32),
            pltpu.VMEM((_PH0, _KE), jnp.int32),
            pltpu.VMEM((_KE, _D), jnp.float32),
            pltpu.VMEM((_KE, _D), jnp.float32),
            pltpu.VMEM((_KE, _D), jnp.float32),
            pltpu.VMEM((16, _D), jnp.float32),
            pltpu.VMEM_SHARED((_N, _D), jnp.float32),
            pltpu.SemaphoreType.DMA,
            pltpu.SemaphoreType.DMA,
            pltpu.SemaphoreType.DMA,
            pltpu.SemaphoreType.DMA,
            pltpu.SemaphoreType.DMA,
            pltpu.SemaphoreType.DMA,
        ],
    )
    return sc_degree, sc_edge_scatter


def _dinv(dA_ref, dB_ref):
    deg = dA_ref[...] + dB_ref[...] + 1.0
    return lax.rsqrt(jnp.maximum(deg, 1.0))


def _mm_scale_body(x_ref, dA_ref, dB_ref, w_ref, y_ref):
    y_ref[...] = _dinv(dA_ref, dB_ref) * jnp.dot(
        x_ref[...], w_ref[...], preferred_element_type=jnp.float32)


def _mm_scale(x, degp, w):
    return pl.pallas_call(
        _mm_scale_body,
        grid=(_NBLK,),
        in_specs=[
            pl.BlockSpec((_BLK, _D), lambda i: (i, 0)),
            pl.BlockSpec((_BLK, 1), lambda i: (i, 0)),
            pl.BlockSpec((_BLK, 1), lambda i: (i + _NBLK, 0)),
            pl.BlockSpec((_D, _D), lambda i: (0, 0)),
        ],
        out_specs=pl.BlockSpec((_BLK, _D), lambda i: (i, 0)),
        out_shape=jax.ShapeDtypeStruct((_N, _D), jnp.float32),
    )(x, degp, degp, w)


def _fuse_body(zA_ref, zB_ref, y_ref, dA_ref, dB_ref, b_ref, w_ref, o_ref):
    dinv = _dinv(dA_ref, dB_ref)
    sconv = zA_ref[...] + zB_ref[...] + y_ref[...]
    h = jnp.maximum(dinv * sconv + b_ref[...], 0.0)
    o_ref[...] = dinv * jnp.dot(h, w_ref[...],
                                preferred_element_type=jnp.float32)


def _fuse(z, y, degp, b, w):
    return pl.pallas_call(
        _fuse_body,
        grid=(_NBLK,),
        in_specs=[
            pl.BlockSpec((_BLK, _D), lambda i: (i, 0)),
            pl.BlockSpec((_BLK, _D), lambda i: (i + _NBLK, 0)),
            pl.BlockSpec((_BLK, _D), lambda i: (i, 0)),
            pl.BlockSpec((_BLK, 1), lambda i: (i, 0)),
            pl.BlockSpec((_BLK, 1), lambda i: (i + _NBLK, 0)),
            pl.BlockSpec((1, _D), lambda i: (0, 0)),
            pl.BlockSpec((_D, _D), lambda i: (0, 0)),
        ],
        out_specs=pl.BlockSpec((_BLK, _D), lambda i: (i, 0)),
        out_shape=jax.ShapeDtypeStruct((_N, _D), jnp.float32),
    )(z, z, y, degp, degp, b, w)


def _final_body(zA_ref, zB_ref, y_ref, dA_ref, dB_ref, b_ref, bt_ref,
                wf_ref, bf_ref, o_ref, acc, cnt):
    i = pl.program_id(0)

    @pl.when(i == 0)
    def _():
        acc[...] = jnp.zeros_like(acc)
        cnt[...] = jnp.zeros_like(cnt)

    dinv = _dinv(dA_ref, dB_ref)
    sconv = zA_ref[...] + zB_ref[...] + y_ref[...]
    h = jnp.maximum(dinv * sconv + b_ref[...], 0.0)
    pt = (bt_ref[...] == lax.broadcasted_iota(
        jnp.int32, (_BLK, _G), 1)).astype(jnp.float32)
    dn = (((0,), (0,)), ((), ()))
    acc[...] += lax.dot_general(pt, h, dn,
                                preferred_element_type=jnp.float32)
    cnt[...] += lax.dot_general(pt, jnp.ones_like(h), dn,
                                preferred_element_type=jnp.float32)

    @pl.when(i == _NBLK - 1)
    def _():
        pooled = acc[...] / jnp.maximum(cnt[...], 1.0)
        o_ref[...] = jnp.maximum(
            jnp.dot(pooled, wf_ref[...],
                    preferred_element_type=jnp.float32) + bf_ref[...], 0.0)


def _final(z, y, degp, b, bt, wf, bf):
    return pl.pallas_call(
        _final_body,
        grid=(_NBLK,),
        in_specs=[
            pl.BlockSpec((_BLK, _D), lambda i: (i, 0)),
            pl.BlockSpec((_BLK, _D), lambda i: (i + _NBLK, 0)),
            pl.BlockSpec((_BLK, _D), lambda i: (i, 0)),
            pl.BlockSpec((_BLK, 1), lambda i: (i, 0)),
            pl.BlockSpec((_BLK, 1), lambda i: (i + _NBLK, 0)),
            pl.BlockSpec((1, _D), lambda i: (0, 0)),
            pl.BlockSpec((_BLK, 1), lambda i: (i, 0)),
            pl.BlockSpec((_D, _D), lambda i: (0, 0)),
            pl.BlockSpec((1, _D), lambda i: (0, 0)),
        ],
        out_specs=pl.BlockSpec((_G, _D), lambda i: (0, 0)),
        out_shape=jax.ShapeDtypeStruct((_G, _D), jnp.float32),
        scratch_shapes=[
            pltpu.VMEM((_G, _D), jnp.float32),
            pltpu.VMEM((_G, _D), jnp.float32),
        ],
    )(z, z, y, degp, degp, b, bt, wf, bf)


def kernel(x, edge_index, batch, W1, b1, W2, b2, Wf, bf):
    sc_degree, sc_edge_scatter = _sc_kernels()
    src = edge_index[0].astype(jnp.int32)
    dst = edge_index[1].astype(jnp.int32)
    dst2 = dst.reshape(_NC * _NS, _NCHE, _KE)
    degp = sc_degree(dst2).reshape(_NC * _N, 1)
    y1 = _mm_scale(x, degp, W1)
    z1 = sc_edge_scatter(y1, src, dst2)
    y2 = _fuse(z1, y1, degp, b1.reshape(1, _D), W2)
    z2 = sc_edge_scatter(y2, src, dst2)
    return _final(z2, y2, degp, b2.reshape(1, _D),
                  batch.reshape(_N, 1).astype(jnp.int32),
                  Wf, bf.reshape(1, _D))


# speedup vs baseline: 30.2846x; 1.0081x over previous
import functools

import jax
import jax.numpy as jnp
from jax import lax
from jax.experimental import pallas as pl
from jax.experimental.pallas import tpu as pltpu
from jax.experimental.pallas import tpu_sc as plsc

_N = 10000
_D = 128
_E = 320000
_G = 64

_NC = 2
_NS = 16
_EP = _E // (_NC * _NS)
_K = 80
_NCH = _EP // _K
_KE = 80
_NCHE = _EP // _KE
_PH0 = 64
_RPT = 624
_ZCH = _RPT // 16

_BLK = 1000
_NBLK = _N // _BLK


def _sc_mesh():
    return plsc.VectorSubcoreMesh(core_axis_name="c", subcore_axis_name="s",
                                  num_cores=_NC, num_subcores=_NS)


def _sc_degree_body(dst_hbm, out_hbm, didx, ones, zbuf, fbuf, dacc, sem):
    c = lax.axis_index("c")
    s = lax.axis_index("s")
    row0 = s * _RPT
    for q in range(16):
        zbuf[pl.ds(q * 16, 16)] = jnp.zeros((16,), jnp.float32)
    for q in range(_K // 16):
        ones[pl.ds(q * 16, 16)] = jnp.ones((16,), jnp.float32)

    def _zero(i, carry):
        pltpu.async_copy(zbuf, dacc.at[pl.ds(row0 + i * 256, 256)], sem)
        return carry

    lax.fori_loop(0, _RPT // 256, _zero, 0)
    pltpu.async_copy(zbuf.at[pl.ds(0, _RPT % 256)],
                     dacc.at[pl.ds(row0 + _RPT - _RPT % 256, _RPT % 256)], sem)

    @pl.when(s == _NS - 1)
    def _zero_tail():
        pltpu.async_copy(zbuf.at[pl.ds(0, 16)], dacc.at[pl.ds(_N - 16, 16)],
                         sem)

    def _zero_drain(i, carry):
        pltpu.make_async_copy(zbuf, dacc.at[pl.ds(row0, 256)], sem).wait()
        return carry

    lax.fori_loop(0, _RPT // 256, _zero_drain, 0)
    pltpu.make_async_copy(zbuf.at[pl.ds(0, _RPT % 256)],
                          dacc.at[pl.ds(row0, _RPT % 256)], sem).wait()

    @pl.when(s == _NS - 1)
    def _zero_drain_tail():
        pltpu.make_async_copy(zbuf.at[pl.ds(0, 16)], dacc.at[pl.ds(row0, 16)],
                              sem).wait()

    plsc.subcore_barrier()

    wid = c * _NS + s
    pltpu.sync_copy(dst_hbm.at[wid], didx)

    def _fire(i, carry):
        pltpu.async_copy(ones, dacc.at[didx.at[i]], sem, add=True)
        return carry

    lax.fori_loop(0, _NCH, _fire, 0)

    def _drain(i, carry):
        pltpu.make_async_copy(ones, dacc.at[didx.at[0]], sem).wait()
        return carry

    lax.fori_loop(0, _NCH, _drain, 0)
    plsc.subcore_barrier()
    pltpu.sync_copy(dacc.at[pl.ds(row0, _RPT)], fbuf.at[pl.ds(0, _RPT)])
    pltpu.sync_copy(fbuf.at[pl.ds(0, _RPT)],
                    out_hbm.at[pl.ds(c * _N + row0, _RPT)])

    @pl.when(s == _NS - 1)
    def _flush_tail():
        pltpu.sync_copy(dacc.at[pl.ds(_NS * _RPT, _N - _NS * _RPT)],
                        fbuf.at[pl.ds(0, _N - _NS * _RPT)])
        pltpu.sync_copy(fbuf.at[pl.ds(0, _N - _NS * _RPT)],
                        out_hbm.at[pl.ds(c * _N + _NS * _RPT,
                                         _N - _NS * _RPT)])


def _sc_edge_scatter_body(y_hbm, src_hbm, dst_hbm, out_hbm,
                          sidx, didx, rows0, rows1, rows2, zbuf, zacc,
                          sg0, sg1, sg2, ss0, ss1, ss2):
    c = lax.axis_index("c")
    s = lax.axis_index("s")
    row0 = s * _RPT
    for r in range(16):
        for q in range(_D // 16):
            zbuf[r, pl.ds(q * 16, 16)] = jnp.zeros((16,), jnp.float32)

    def _zero(i, carry):
        pltpu.async_copy(zbuf, zacc.at[pl.ds(row0 + i * 16, 16)], sg0)
        return carry

    lax.fori_loop(0, _ZCH, _zero, 0)

    @pl.when(s == _NS - 1)
    def _zero_tail():
        pltpu.async_copy(zbuf, zacc.at[pl.ds(_N - 16, 16)], sg0)

    def _zero_drain(i, carry):
        pltpu.make_async_copy(zbuf, zacc.at[pl.ds(row0, 16)], sg0).wait()
        return carry

    lax.fori_loop(0, _ZCH, _zero_drain, 0)

    @pl.when(s == _NS - 1)
    def _zero_drain_tail():
        pltpu.make_async_copy(zbuf, zacc.at[pl.ds(row0, 16)], sg0).wait()

    plsc.subcore_barrier()

    wid = c * _NS + s
    rows_ = (rows0, rows1, rows2)
    sg = (sg0, sg1, sg2)
    ss = (ss0, ss1, ss2)

    def _gather(j, b):
        pltpu.async_copy(y_hbm.at[sidx.at[pl.ds(j * _KE, _KE)]],
                         rows_[b], sg[b])

    def _wait_g(b):
        pltpu.make_async_copy(y_hbm.at[sidx.at[pl.ds(0, _KE)]],
                              rows_[b], sg[b]).wait()

    def _scatter(j, b):
        pltpu.async_copy(rows_[b], zacc.at[didx.at[j]], ss[b], add=True)

    def _wait_s(b):
        pltpu.make_async_copy(rows_[b], zacc.at[didx.at[0]], ss[b]).wait()

    for ph, nch in ((0, _PH0), (1, _NCHE - _PH0)):
        g0 = ph * _PH0
        pltpu.sync_copy(src_hbm.at[pl.ds(wid * _EP + g0 * _KE, nch * _KE)],
                        sidx.at[pl.ds(0, nch * _KE)])
        pltpu.sync_copy(dst_hbm.at[wid, pl.ds(g0, nch)],
                        didx.at[pl.ds(0, nch)])
        nt = nch // 3
        for b in range(3):
            _gather(b, b)
        for b in range(3):
            _wait_g(b)
            _scatter(b, b)

        def _body(t, carry):
            j = 3 * t
            for b in range(3):
                _wait_s(b)
                _gather(j + b, b)
            for b in range(3):
                _wait_g(b)
                _scatter(j + b, b)
            return carry

        lax.fori_loop(1, nt, _body, 0)
        for j in range(3 * nt, nch):
            b = j - 3 * nt
            _wait_s(b)
            _gather(j, b)
            _wait_g(b)
            _scatter(j, b)
        for b in range(3):
            _wait_s(b)

    plsc.subcore_barrier()
    pltpu.sync_copy(zacc.at[pl.ds(row0, _RPT)],
                    out_hbm.at[pl.ds(c * _N + row0, _RPT)])

    @pl.when(s == _NS - 1)
    def _flush_tail():
        pltpu.sync_copy(zacc.at[pl.ds(_NS * _RPT, _N - _NS * _RPT)],
                        out_hbm.at[pl.ds(c * _N + _NS * _RPT,
                                         _N - _NS * _RPT)])


@functools.cache
def _sc_kernels():
    sc_degree = pl.kernel(
        _sc_degree_body,
        out_type=jax.ShapeDtypeStruct((_NC * _N,), jnp.float32),
        mesh=_sc_mesh(),
        scratch_types=[
            pltpu.VMEM((_NCH, _K), jnp.int32),
            pltpu.VMEM((_K,), jnp.float32),
            pltpu.VMEM((256,), jnp.float32),
            pltpu.VMEM((_RPT,), jnp.float32),
            pltpu.VMEM_SHARED((_N,), jnp.float32),
            pltpu.SemaphoreType.DMA,
        ],
    )
    sc_edge_scatter = pl.kernel(
        _sc_edge_scatter_body,
        out_type=jax.ShapeDtypeStruct((_NC * _N, _D), jnp.float32),
        mesh=_sc_mesh(),
        scratch_types=[
            pltpu.VMEM((_PH0 * _KE,), jnp.int32),
            pltpu.VMEM((_PH0, _KE), jnp.int32),
            pltpu.VMEM((_KE, _D), jnp.float32),
            pltpu.VMEM((_KE, _D), jnp.float32),
            pltpu.VMEM((_KE, _D), jnp.float32),
            pltpu.VMEM((16, _D), jnp.float32),
            pltpu.VMEM_SHARED((_N, _D), jnp.float32),
            pltpu.SemaphoreType.DMA,
            pltpu.SemaphoreType.DMA,
            pltpu.SemaphoreType.DMA,
            pltpu.SemaphoreType.DMA,
            pltpu.SemaphoreType.DMA,
            pltpu.SemaphoreType.DMA,
        ],
    )
    return sc_degree, sc_edge_scatter


def _dinv(dA_ref, dB_ref):
    deg = dA_ref[...] + dB_ref[...] + 1.0
    return lax.rsqrt(jnp.maximum(deg, 1.0))


def _mm_scale_body(x_ref, dA_ref, dB_ref, w_ref, y_ref):
    y_ref[...] = _dinv(dA_ref, dB_ref) * jnp.dot(
        x_ref[...], w_ref[...], preferred_element_type=jnp.float32)


def _mm_scale(x, degp, w):
    return pl.pallas_call(
        _mm_scale_body,
        grid=(_NBLK,),
        in_specs=[
            pl.BlockSpec((_BLK, _D), lambda i: (i, 0)),
            pl.BlockSpec((_BLK, 1), lambda i: (i, 0)),
            pl.BlockSpec((_BLK, 1), lambda i: (i + _NBLK, 0)),
            pl.BlockSpec((_D, _D), lambda i: (0, 0)),
        ],
        out_specs=pl.BlockSpec((_BLK, _D), lambda i: (i, 0)),
        out_shape=jax.ShapeDtypeStruct((_N, _D), jnp.float32),
    )(x, degp, degp, w)


def _fuse_body(zA_ref, zB_ref, y_ref, dA_ref, dB_ref, b_ref, w_ref, o_ref):
    dinv = _dinv(dA_ref, dB_ref)
    sconv = zA_ref[...] + zB_ref[...] + y_ref[...]
    h = jnp.maximum(dinv * sconv + b_ref[...], 0.0)
    o_ref[...] = dinv * jnp.dot(h, w_ref[...],
                                preferred_element_type=jnp.float32)


def _fuse(z, y, degp, b, w):
    return pl.pallas_call(
        _fuse_body,
        grid=(_NBLK,),
        in_specs=[
            pl.BlockSpec((_BLK, _D), lambda i: (i, 0)),
            pl.BlockSpec((_BLK, _D), lambda i: (i + _NBLK, 0)),
            pl.BlockSpec((_BLK, _D), lambda i: (i, 0)),
            pl.BlockSpec((_BLK, 1), lambda i: (i, 0)),
            pl.BlockSpec((_BLK, 1), lambda i: (i + _NBLK, 0)),
            pl.BlockSpec((1, _D), lambda i: (0, 0)),
            pl.BlockSpec((_D, _D), lambda i: (0, 0)),
        ],
        out_specs=pl.BlockSpec((_BLK, _D), lambda i: (i, 0)),
        out_shape=jax.ShapeDtypeStruct((_N, _D), jnp.float32),
    )(z, z, y, degp, degp, b, w)


def _final_body(zA_ref, zB_ref, y_ref, dA_ref, dB_ref, b_ref, bt_ref,
                wf_ref, bf_ref, o_ref, acc, cnt):
    i = pl.program_id(0)

    @pl.when(i == 0)
    def _():
        acc[...] = jnp.zeros_like(acc)
        cnt[...] = jnp.zeros_like(cnt)

    dinv = _dinv(dA_ref, dB_ref)
    sconv = zA_ref[...] + zB_ref[...] + y_ref[...]
    h = jnp.maximum(dinv * sconv + b_ref[...], 0.0)
    pt = (bt_ref[...] == lax.broadcasted_iota(
        jnp.int32, (_BLK, _G), 1)).astype(jnp.float32)
    dn = (((0,), (0,)), ((), ()))
    acc[...] += lax.dot_general(pt, h, dn,
                                preferred_element_type=jnp.float32)
    cnt[...] += lax.dot_general(pt, jnp.ones_like(h), dn,
                                preferred_element_type=jnp.float32)

    @pl.when(i == _NBLK - 1)
    def _():
        pooled = acc[...] / jnp.maximum(cnt[...], 1.0)
        o_ref[...] = jnp.maximum(
            jnp.dot(pooled, wf_ref[...],
                    preferred_element_type=jnp.float32) + bf_ref[...], 0.0)


def _final(z, y, degp, b, bt, wf, bf):
    return pl.pallas_call(
        _final_body,
        grid=(_NBLK,),
        in_specs=[
            pl.BlockSpec((_BLK, _D), lambda i: (i, 0)),
            pl.BlockSpec((_BLK, _D), lambda i: (i + _NBLK, 0)),
            pl.BlockSpec((_BLK, _D), lambda i: (i, 0)),
            pl.BlockSpec((_BLK, 1), lambda i: (i, 0)),
            pl.BlockSpec((_BLK, 1), lambda i: (i + _NBLK, 0)),
            pl.BlockSpec((1, _D), lambda i: (0, 0)),
            pl.BlockSpec((_BLK, 1), lambda i: (i, 0)),
            pl.BlockSpec((_D, _D), lambda i: (0, 0)),
            pl.BlockSpec((1, _D), lambda i: (0, 0)),
        ],
        out_specs=pl.BlockSpec((_G, _D), lambda i: (0, 0)),
        out_shape=jax.ShapeDtypeStruct((_G, _D), jnp.float32),
        scratch_shapes=[
            pltpu.VMEM((_G, _D), jnp.float32),
            pltpu.VMEM((_G, _D), jnp.float32),
        ],
    )(z, z, y, degp, degp, b, bt, wf, bf)


def kernel(x, edge_index, batch, W1, b1, W2, b2, Wf, bf):
    sc_degree, sc_edge_scatter = _sc_kernels()
    src = edge_index[0].astype(jnp.int32)
    dst = edge_index[1].astype(jnp.int32)
    dst2 = dst.reshape(_NC * _NS, _NCHE, _KE)
    degp = sc_degree(dst2).reshape(_NC * _N, 1)
    y1 = _mm_scale(x, degp, W1)
    z1 = sc_edge_scatter(y1, src, dst2)
    y2 = _fuse(z1, y1, degp, b1.reshape(1, _D), W2)
    z2 = sc_edge_scatter(y2, src, dst2)
    return _final(z2, y2, degp, b2.reshape(1, _D),
                  batch.reshape(_N, 1).astype(jnp.int32),
                  Wf, bf.reshape(1, _D))
